# baseline (speedup 1.0000x reference)
import functools

import numpy as np
import jax
import jax.numpy as jnp
from jax import lax
from jax.experimental import pallas as pl
from jax.experimental.pallas import tpu as pltpu
from jax.experimental.pallas import tpu_sc as plsc

_B = 32
_NPL = [1024, 2048, 2048, 2048]
_MPL = [2048, 2048, 2048, 512]
_W16 = 16

_M32 = np.uint64(0xFFFFFFFF)
_ROT = [[13, 15, 26, 6], [17, 29, 16, 24]]


def _tf2x32_np(k0, k1, x0, x1):
    k0 = np.uint64(k0)
    k1 = np.uint64(k1)
    ks = [k0, k1, (k0 ^ k1 ^ np.uint64(0x1BD11BDA)) & _M32]
    x0 = (np.uint64(x0) + ks[0]) & _M32
    x1 = (np.uint64(x1) + ks[1]) & _M32
    for i in range(5):
        for r in _ROT[i % 2]:
            x0 = (x0 + x1) & _M32
            x1 = ((x1 << np.uint64(r)) | (x1 >> np.uint64(32 - r))) & _M32
            x1 = x1 ^ x0
        x0 = (x0 + ks[(i + 1) % 3]) & _M32
        x1 = (x1 + ks[(i + 2) % 3] + np.uint64(i + 1)) & _M32
    return int(x0), int(x1)


def _derive_keys():
    keys = []
    base = (0, 42)
    for i in range(4):
        kl = _tf2x32_np(base[0], base[1], 0, i)
        k1 = _tf2x32_np(kl[0], kl[1], 0, 0)
        k2 = _tf2x32_np(kl[0], kl[1], 0, 1)
        keys.append((k1, k2))
    return keys


_KEYS = _derive_keys()

_TINY = np.float32(np.finfo(np.float32).tiny)
_ONEBITS = np.uint32(np.float32(1.0).view(np.uint32))


def _threefry_bits(key, x1):
    k0 = jnp.uint32(key[0])
    k1 = jnp.uint32(key[1])
    k2 = jnp.uint32((key[0] ^ key[1] ^ 0x1BD11BDA) & 0xFFFFFFFF)
    ks = [k0, k1, k2]
    x0 = jnp.full(x1.shape, k0, jnp.uint32)
    x1 = x1 + k1
    for i in range(5):
        for r in _ROT[i % 2]:
            x0 = x0 + x1
            x1 = (x1 << r) | (x1 >> (32 - r))
            x1 = x1 ^ x0
        x0 = x0 + ks[(i + 1) % 3]
        x1 = x1 + ks[(i + 2) % 3] + jnp.uint32(i + 1)
    return x0 ^ x1


def _bits_to_f01(bits):
    fb = (bits >> jnp.uint32(9)) | _ONEBITS
    return lax.bitcast_convert_type(fb, jnp.float32) - jnp.float32(1.0)


def _pack_body(x_ref, o_ref):
    x = x_ref[...]
    sh = lax.broadcasted_iota(jnp.int32, (x.shape[0], 32), 1)
    cols = []
    for j in range(8):
        w = x[:, 32 * j:32 * (j + 1)]
        cols.append(jnp.sum(w << sh, axis=1)[:, None])
    l = jnp.concatenate(cols, axis=1)
    o_ref[...] = jnp.concatenate([l, jnp.zeros_like(l)], axis=1)


def _pack(x):
    return pl.pallas_call(
        _pack_body,
        out_shape=jax.ShapeDtypeStruct((x.shape[0], _W16), jnp.int32),
    )(x)


def _make_sample(M, N, off_mult, k1, k2, rb):
    grid_j = (2 * M) // rb

    def body(logits_ref, p_ref, samp_ref, inv_ref):
        j = pl.program_id(0)
        b = pl.program_id(1)
        base = ((b * (2 * M) + j * rb) * N).astype(jnp.uint32)
        row = lax.broadcasted_iota(jnp.uint32, (rb, N), 0)
        col = lax.broadcasted_iota(jnp.uint32, (rb, N), 1)
        cnt = base + row * jnp.uint32(N) + col
        f = _bits_to_f01(_threefry_bits(k1, cnt))
        u = jnp.maximum(_TINY, f * (jnp.float32(1.0) - _TINY) + _TINY)
        g = -jnp.log(-jnp.log(u))
        v = g + logits_ref[...]
        rowmax = jnp.max(v, axis=1, keepdims=True)
        coli = lax.broadcasted_iota(jnp.int32, (rb, N), 1)
        cand = jnp.where(v == rowmax, coli, jnp.int32(N))
        samp = jnp.min(cand, axis=1) + b * jnp.int32(off_mult)
        samp_ref[...] = samp.reshape(1, 1, 1, rb)

        @pl.when(jnp.logical_and(j == 0, b == 0))
        def _():
            mi = lax.broadcasted_iota(jnp.uint32, (M, _W16), 0)
            f2 = _bits_to_f01(_threefry_bits(k2, mi))
            u2 = jnp.maximum(jnp.float32(0.0), f2)
            invw = jnp.where(u2 < p_ref[...], jnp.int32(-1), jnp.int32(0))
            inv_ref[...] = invw

    call = pl.pallas_call(
        body,
        grid=(grid_j, _B),
        in_specs=[
            pl.BlockSpec((rb, N), lambda j, b: (j, 0)),
            pl.BlockSpec((M, 1), lambda j, b: (0, 0)),
        ],
        out_specs=[
            pl.BlockSpec((1, 1, 1, rb), lambda j, b: (b, j, 0, 0)),
            pl.BlockSpec((M, _W16), lambda j, b: (0, 0)),
        ],
        out_shape=[
            jax.ShapeDtypeStruct((_B, grid_j, 1, rb), jnp.int32),
            jax.ShapeDtypeStruct((M, _W16), jnp.int32),
        ],
    )

    def run(logits2d, p_col):
        samp4, invw = call(logits2d, p_col)
        return samp4.reshape(_B, 2 * M), invw

    return run


_CH = 128


def _make_sc_layer(M):
    n_ch = M // _CH
    mesh = plsc.VectorSubcoreMesh(core_axis_name="c", subcore_axis_name="s")

    @functools.partial(
        pl.kernel,
        mesh=mesh,
        compiler_params=pltpu.CompilerParams(use_tc_tiling_on_sc=False),
        out_type=jax.ShapeDtypeStruct((_B, M, _W16), jnp.int32),
        scratch_types=[
            pltpu.VMEM((2, _CH), jnp.int32),
            pltpu.VMEM((_CH, _W16), jnp.int32),
            pltpu.VMEM((_CH, _W16), jnp.int32),
            pltpu.VMEM((_CH, _W16), jnp.int32),
            pltpu.VMEM((_CH, _W16), jnp.int32),
            pltpu.SemaphoreType.DMA,
            pltpu.SemaphoreType.DMA,
        ],
    )
    def k(src_hbm, samp_hbm, inv_hbm, out_hbm,
          idx_s, rows0, rows1, invv, outv, sem0, sem1):
        b = lax.axis_index("s") * 2 + lax.axis_index("c")

        def chunk(ci, carry):
            m0 = ci * _CH
            pltpu.sync_copy(samp_hbm.at[b, pl.ds(m0, _CH)], idx_s.at[0])
            pltpu.sync_copy(samp_hbm.at[b, pl.ds(M + m0, _CH)], idx_s.at[1])
            cp0 = pltpu.async_copy(src_hbm.at[idx_s.at[0]], rows0, sem0)
            cp1 = pltpu.async_copy(src_hbm.at[idx_s.at[1]], rows1, sem1)
            pltpu.sync_copy(inv_hbm.at[pl.ds(m0, _CH)], invv)
            cp0.wait()
            cp1.wait()

            def gate(i, c):
                a = rows0[i, :] & rows1[i, :]
                outv[i, :] = a ^ invv[i, :]
                return c

            lax.fori_loop(0, _CH, gate, 0)
            pltpu.sync_copy(outv, out_hbm.at[b, pl.ds(m0, _CH)])
            return carry

        lax.fori_loop(0, n_ch, chunk, 0)

    return k


def _unpack_body(y_ref, o_ref):
    y = y_ref[...]
    sh = lax.broadcasted_iota(jnp.int32, (y.shape[0], 32), 1)
    for j in range(8):
        lb = (y[:, j:j + 1] >> sh) & 1
        hb = (y[:, j + 8:j + 9] >> sh) & 1
        o_ref[:, 32 * j:32 * (j + 1)] = lb - 2 * hb


def _unpack(y, rb=2048):
    rows = y.shape[0]
    return pl.pallas_call(
        _unpack_body,
        grid=(rows // rb,),
        in_specs=[pl.BlockSpec((rb, _W16), lambda r: (r, 0))],
        out_specs=pl.BlockSpec((rb, 256), lambda r: (r, 0)),
        out_shape=jax.ShapeDtypeStruct((rows, 256), jnp.int32),
    )(y)


_SAMPLERS = [
    _make_sample(_MPL[l], _NPL[l], 0 if l == 0 else _NPL[l],
                 _KEYS[l][0], _KEYS[l][1], rb=1024)
    for l in range(4)
]
_SC_LAYERS = [_make_sc_layer(_MPL[l]) for l in range(4)]


def kernel(input_bitarrays, batch_size,
           adj_logits_0, invert_logits_0,
           adj_logits_1, invert_logits_1,
           adj_logits_2, invert_logits_2,
           adj_logits_3, invert_logits_3):
    adjs = [adj_logits_0, adj_logits_1, adj_logits_2, adj_logits_3]
    invs = [invert_logits_0, invert_logits_1, invert_logits_2, invert_logits_3]

    samps, invws = [], []
    for l in range(4):
        M, N = _MPL[l], _NPL[l]
        logits2d = adjs[l].reshape(2 * M, N)
        p_col = jax.nn.sigmoid(invs[l]).reshape(M, 1)
        s, iw = _SAMPLERS[l](logits2d, p_col)
        samps.append(s)
        invws.append(iw)

    x = _pack(input_bitarrays)
    for l in range(4):
        y = _SC_LAYERS[l](x, samps[l], invws[l])
        x = y.reshape(_B * _MPL[l], _W16)

    out = _unpack(x)
    return out.reshape(_B, _MPL[3], 256) + (batch_size - batch_size)


# speedup vs baseline: 1.4262x; 1.4262x over previous
import functools

import numpy as np
import jax
import jax.numpy as jnp
from jax import lax
from jax.experimental import pallas as pl
from jax.experimental.pallas import tpu as pltpu
from jax.experimental.pallas import tpu_sc as plsc

_B = 32
_NPL = [1024, 2048, 2048, 2048]
_MPL = [2048, 2048, 2048, 512]
_W16 = 16

_M32 = np.uint64(0xFFFFFFFF)
_ROT = [[13, 15, 26, 6], [17, 29, 16, 24]]


def _tf2x32_np(k0, k1, x0, x1):
    k0 = np.uint64(k0)
    k1 = np.uint64(k1)
    ks = [k0, k1, (k0 ^ k1 ^ np.uint64(0x1BD11BDA)) & _M32]
    x0 = (np.uint64(x0) + ks[0]) & _M32
    x1 = (np.uint64(x1) + ks[1]) & _M32
    for i in range(5):
        for r in _ROT[i % 2]:
            x0 = (x0 + x1) & _M32
            x1 = ((x1 << np.uint64(r)) | (x1 >> np.uint64(32 - r))) & _M32
            x1 = x1 ^ x0
        x0 = (x0 + ks[(i + 1) % 3]) & _M32
        x1 = (x1 + ks[(i + 2) % 3] + np.uint64(i + 1)) & _M32
    return int(x0), int(x1)


def _derive_keys():
    keys = []
    base = (0, 42)
    for i in range(4):
        kl = _tf2x32_np(base[0], base[1], 0, i)
        k1 = _tf2x32_np(kl[0], kl[1], 0, 0)
        k2 = _tf2x32_np(kl[0], kl[1], 0, 1)
        keys.append((k1, k2))
    return keys


_KEYS = _derive_keys()

_TINY = np.float32(np.finfo(np.float32).tiny)
_ONEBITS = np.uint32(np.float32(1.0).view(np.uint32))


def _threefry_bits(key, x1):
    k0 = jnp.uint32(key[0])
    k1 = jnp.uint32(key[1])
    k2 = jnp.uint32((key[0] ^ key[1] ^ 0x1BD11BDA) & 0xFFFFFFFF)
    ks = [k0, k1, k2]
    x0 = jnp.full(x1.shape, k0, jnp.uint32)
    x1 = x1 + k1
    for i in range(5):
        for r in _ROT[i % 2]:
            x0 = x0 + x1
            x1 = (x1 << r) | (x1 >> (32 - r))
            x1 = x1 ^ x0
        x0 = x0 + ks[(i + 1) % 3]
        x1 = x1 + ks[(i + 2) % 3] + jnp.uint32(i + 1)
    return x0 ^ x1


def _bits_to_f01(bits):
    fb = (bits >> jnp.uint32(9)) | _ONEBITS
    return lax.bitcast_convert_type(fb, jnp.float32) - jnp.float32(1.0)


def _pack_body(x_ref, o_ref):
    x = x_ref[...]
    sh = lax.broadcasted_iota(jnp.int32, (x.shape[0], 32), 1)
    cols = []
    for j in range(8):
        w = x[:, 32 * j:32 * (j + 1)]
        cols.append(jnp.sum(w << sh, axis=1)[:, None])
    l = jnp.concatenate(cols, axis=1)
    o_ref[...] = jnp.concatenate([l, jnp.zeros_like(l)], axis=1)


def _pack(x):
    return pl.pallas_call(
        _pack_body,
        out_shape=jax.ShapeDtypeStruct((x.shape[0], _W16), jnp.int32),
    )(x)


def _make_sample(M, N, off_mult, k1, k2, rb, b0=0, nb=_B):
    grid_j = (2 * M) // rb

    def body(logits_ref, p_ref, samp_ref, inv_ref):
        j = pl.program_id(0)
        b = pl.program_id(1) + b0
        base = ((b * (2 * M) + j * rb) * N).astype(jnp.uint32)
        row = lax.broadcasted_iota(jnp.uint32, (rb, N), 0)
        col = lax.broadcasted_iota(jnp.uint32, (rb, N), 1)
        cnt = base + row * jnp.uint32(N) + col
        f = _bits_to_f01(_threefry_bits(k1, cnt))
        u = jnp.maximum(_TINY, f * (jnp.float32(1.0) - _TINY) + _TINY)
        g = -jnp.log(-jnp.log(u))
        v = g + logits_ref[...]
        rowmax = jnp.max(v, axis=1, keepdims=True)
        coli = lax.broadcasted_iota(jnp.int32, (rb, N), 1)
        cand = jnp.where(v == rowmax, coli, jnp.int32(N))
        samp = jnp.min(cand, axis=1) + b * jnp.int32(off_mult)
        samp_ref[...] = samp.reshape(1, 1, 1, rb)

        @pl.when(jnp.logical_and(j == 0, pl.program_id(1) == 0))
        def _():
            mi = lax.broadcasted_iota(jnp.uint32, (M, _W16), 0)
            f2 = _bits_to_f01(_threefry_bits(k2, mi))
            u2 = jnp.maximum(jnp.float32(0.0), f2)
            invw = jnp.where(u2 < p_ref[...], jnp.int32(-1), jnp.int32(0))
            inv_ref[...] = invw

    call = pl.pallas_call(
        body,
        grid=(grid_j, nb),
        in_specs=[
            pl.BlockSpec((rb, N), lambda j, b: (j, 0)),
            pl.BlockSpec((M, 1), lambda j, b: (0, 0)),
        ],
        out_specs=[
            pl.BlockSpec((1, 1, 1, rb), lambda j, b: (b, j, 0, 0)),
            pl.BlockSpec((M, _W16), lambda j, b: (0, 0)),
        ],
        out_shape=[
            jax.ShapeDtypeStruct((nb, grid_j, 1, rb), jnp.int32),
            jax.ShapeDtypeStruct((M, _W16), jnp.int32),
        ],
    )

    def run(logits2d, p_col):
        samp4, invw = call(logits2d, p_col)
        return samp4.reshape(nb, 2 * M), invw

    return run


_BSC = 8
_BCHUNK = 16384
_BUN = 8


def _make_sc_bits(rows, N, key):
    sz = rows * N
    szw = sz // 32
    n_outer = szw // _BCHUNK
    n_inner = _BCHUNK // (16 * _BUN)
    mesh = plsc.VectorSubcoreMesh(core_axis_name="c", subcore_axis_name="s")

    @functools.partial(
        pl.kernel,
        mesh=mesh,
        compiler_params=pltpu.CompilerParams(use_tc_tiling_on_sc=False),
        out_type=jax.ShapeDtypeStruct((sz,), jnp.uint32),
        scratch_types=[
            pltpu.VMEM((2, _BCHUNK), jnp.uint32),
            pltpu.SemaphoreType.DMA,
            pltpu.SemaphoreType.DMA,
        ],
    )
    def k(out_hbm, buf, sem0, sem1):
        w = lax.axis_index("s") * 2 + lax.axis_index("c")
        base = w * szw
        lane = lax.iota(jnp.uint32, 16)

        def outer(oi, carry):
            off = base + oi * _BCHUNK
            sel = lax.rem(oi, 2)

            def inner(ii, c2):
                o2 = ii * (16 * _BUN)
                c0 = jnp.uint32(off + o2) + lane
                for u in range(_BUN):
                    bits = _threefry_bits(key, c0 + jnp.uint32(16 * u))
                    buf[sel, pl.ds(o2 + 16 * u, 16)] = bits
                return c2

            @pl.when(oi >= 2)
            def _():
                @pl.when(sel == 0)
                def _():
                    pltpu.make_async_copy(
                        out_hbm.at[pl.ds(0, _BCHUNK)], buf.at[0], sem0).wait()

                @pl.when(sel == 1)
                def _():
                    pltpu.make_async_copy(
                        out_hbm.at[pl.ds(0, _BCHUNK)], buf.at[1], sem1).wait()

            lax.fori_loop(0, n_inner, inner, 0)

            @pl.when(sel == 0)
            def _():
                pltpu.async_copy(buf.at[0], out_hbm.at[pl.ds(off, _BCHUNK)], sem0)

            @pl.when(sel == 1)
            def _():
                pltpu.async_copy(buf.at[1], out_hbm.at[pl.ds(off, _BCHUNK)], sem1)

            return carry

        lax.fori_loop(0, n_outer, outer, 0)
        pltpu.make_async_copy(out_hbm.at[pl.ds(0, _BCHUNK)], buf.at[0], sem0).wait()
        pltpu.make_async_copy(out_hbm.at[pl.ds(0, _BCHUNK)], buf.at[1], sem1).wait()

    return k


def _make_sample_frombits(M, N, off_mult, rb):
    grid_j = (2 * M) // rb
    jpb = (2 * M) // rb

    def body(bits_ref, logits_ref, samp_ref):
        j = pl.program_id(0)
        b = pl.program_id(1)
        f = _bits_to_f01(bits_ref[...])
        u = jnp.maximum(_TINY, f * (jnp.float32(1.0) - _TINY) + _TINY)
        g = -jnp.log(-jnp.log(u))
        v = g + logits_ref[...]
        rowmax = jnp.max(v, axis=1, keepdims=True)
        coli = lax.broadcasted_iota(jnp.int32, (rb, N), 1)
        cand = jnp.where(v == rowmax, coli, jnp.int32(N))
        samp = jnp.min(cand, axis=1) + b * jnp.int32(off_mult)
        samp_ref[...] = samp.reshape(1, 1, 1, rb)

    call = pl.pallas_call(
        body,
        grid=(grid_j, _BSC),
        in_specs=[
            pl.BlockSpec((rb, N), lambda j, b: (b * jpb + j, 0)),
            pl.BlockSpec((rb, N), lambda j, b: (j, 0)),
        ],
        out_specs=pl.BlockSpec((1, 1, 1, rb), lambda j, b: (b, j, 0, 0)),
        out_shape=jax.ShapeDtypeStruct((_BSC, grid_j, 1, rb), jnp.int32),
    )

    def run(bits2d, logits2d):
        return call(bits2d, logits2d).reshape(_BSC, 2 * M)

    return run


_CH = 128


def _make_sc_layer(M):
    n_ch = M // _CH
    mesh = plsc.VectorSubcoreMesh(core_axis_name="c", subcore_axis_name="s")

    @functools.partial(
        pl.kernel,
        mesh=mesh,
        compiler_params=pltpu.CompilerParams(use_tc_tiling_on_sc=False),
        out_type=jax.ShapeDtypeStruct((_B, M, _W16), jnp.int32),
        scratch_types=[
            pltpu.VMEM((2, _CH), jnp.int32),
            pltpu.VMEM((_CH, _W16), jnp.int32),
            pltpu.VMEM((_CH, _W16), jnp.int32),
            pltpu.VMEM((_CH, _W16), jnp.int32),
            pltpu.VMEM((_CH, _W16), jnp.int32),
            pltpu.SemaphoreType.DMA,
            pltpu.SemaphoreType.DMA,
        ],
    )
    def k(src_hbm, samp_hbm, inv_hbm, out_hbm,
          idx_s, rows0, rows1, invv, outv, sem0, sem1):
        b = lax.axis_index("s") * 2 + lax.axis_index("c")

        def chunk(ci, carry):
            m0 = ci * _CH
            pltpu.sync_copy(samp_hbm.at[b, pl.ds(m0, _CH)], idx_s.at[0])
            pltpu.sync_copy(samp_hbm.at[b, pl.ds(M + m0, _CH)], idx_s.at[1])
            cp0 = pltpu.async_copy(src_hbm.at[idx_s.at[0]], rows0, sem0)
            cp1 = pltpu.async_copy(src_hbm.at[idx_s.at[1]], rows1, sem1)
            pltpu.sync_copy(inv_hbm.at[pl.ds(m0, _CH)], invv)
            cp0.wait()
            cp1.wait()

            def gate(i, c):
                a = rows0[i, :] & rows1[i, :]
                outv[i, :] = a ^ invv[i, :]
                return c

            lax.fori_loop(0, _CH, gate, 0)
            pltpu.sync_copy(outv, out_hbm.at[b, pl.ds(m0, _CH)])
            return carry

        lax.fori_loop(0, n_ch, chunk, 0)

    return k


def _unpack_body(y_ref, o_ref):
    y = y_ref[...]
    sh = lax.broadcasted_iota(jnp.int32, (y.shape[0], 32), 1)
    for j in range(8):
        lb = (y[:, j:j + 1] >> sh) & 1
        hb = (y[:, j + 8:j + 9] >> sh) & 1
        o_ref[:, 32 * j:32 * (j + 1)] = lb - 2 * hb


def _unpack(y, rb=2048):
    rows = y.shape[0]
    return pl.pallas_call(
        _unpack_body,
        grid=(rows // rb,),
        in_specs=[pl.BlockSpec((rb, _W16), lambda r: (r, 0))],
        out_specs=pl.BlockSpec((rb, 256), lambda r: (r, 0)),
        out_shape=jax.ShapeDtypeStruct((rows, 256), jnp.int32),
    )(y)


_SAMPLERS = [
    _make_sample(_MPL[l], _NPL[l], 0 if l == 0 else _NPL[l],
                 _KEYS[l][0], _KEYS[l][1], rb=512, b0=_BSC, nb=_B - _BSC)
    for l in range(4)
]
_SC_BITS = [
    _make_sc_bits(_BSC * 2 * _MPL[l], _NPL[l], _KEYS[l][0]) for l in range(4)
]
_SAMPLERS_FB = [
    _make_sample_frombits(_MPL[l], _NPL[l], 0 if l == 0 else _NPL[l], rb=512)
    for l in range(4)
]
_SC_LAYERS = [_make_sc_layer(_MPL[l]) for l in range(4)]


def kernel(input_bitarrays, batch_size,
           adj_logits_0, invert_logits_0,
           adj_logits_1, invert_logits_1,
           adj_logits_2, invert_logits_2,
           adj_logits_3, invert_logits_3):
    adjs = [adj_logits_0, adj_logits_1, adj_logits_2, adj_logits_3]
    invs = [invert_logits_0, invert_logits_1, invert_logits_2, invert_logits_3]

    bits = [_SC_BITS[l]().reshape(_BSC * 2 * _MPL[l], _NPL[l]) for l in range(4)]

    samps, invws = [], []
    for l in range(4):
        M, N = _MPL[l], _NPL[l]
        logits2d = adjs[l].reshape(2 * M, N)
        p_col = jax.nn.sigmoid(invs[l]).reshape(M, 1)
        s_hi, iw = _SAMPLERS[l](logits2d, p_col)
        s_lo = _SAMPLERS_FB[l](bits[l], logits2d)
        samps.append(jnp.concatenate([s_lo, s_hi], axis=0))
        invws.append(iw)

    x = _pack(input_bitarrays)
    for l in range(4):
        y = _SC_LAYERS[l](x, samps[l], invws[l])
        x = y.reshape(_B * _MPL[l], _W16)

    out = _unpack(x)
    return out.reshape(_B, _MPL[3], 256) + (batch_size - batch_size)


# speedup vs baseline: 1.4917x; 1.0460x over previous
import functools

import numpy as np
import jax
import jax.numpy as jnp
from jax import lax
from jax.experimental import pallas as pl
from jax.experimental.pallas import tpu as pltpu
from jax.experimental.pallas import tpu_sc as plsc

_B = 32
_NPL = [1024, 2048, 2048, 2048]
_MPL = [2048, 2048, 2048, 512]
_W16 = 16

_M32 = np.uint64(0xFFFFFFFF)
_ROT = [[13, 15, 26, 6], [17, 29, 16, 24]]


def _tf2x32_np(k0, k1, x0, x1):
    k0 = np.uint64(k0)
    k1 = np.uint64(k1)
    ks = [k0, k1, (k0 ^ k1 ^ np.uint64(0x1BD11BDA)) & _M32]
    x0 = (np.uint64(x0) + ks[0]) & _M32
    x1 = (np.uint64(x1) + ks[1]) & _M32
    for i in range(5):
        for r in _ROT[i % 2]:
            x0 = (x0 + x1) & _M32
            x1 = ((x1 << np.uint64(r)) | (x1 >> np.uint64(32 - r))) & _M32
            x1 = x1 ^ x0
        x0 = (x0 + ks[(i + 1) % 3]) & _M32
        x1 = (x1 + ks[(i + 2) % 3] + np.uint64(i + 1)) & _M32
    return int(x0), int(x1)


def _derive_keys():
    keys = []
    base = (0, 42)
    for i in range(4):
        kl = _tf2x32_np(base[0], base[1], 0, i)
        k1 = _tf2x32_np(kl[0], kl[1], 0, 0)
        k2 = _tf2x32_np(kl[0], kl[1], 0, 1)
        keys.append((k1, k2))
    return keys


_KEYS = _derive_keys()

_TINY = np.float32(np.finfo(np.float32).tiny)
_ONEBITS = np.uint32(np.float32(1.0).view(np.uint32))


def _threefry_bits(key, x1):
    k0 = jnp.uint32(key[0])
    k1 = jnp.uint32(key[1])
    k2 = jnp.uint32((key[0] ^ key[1] ^ 0x1BD11BDA) & 0xFFFFFFFF)
    ks = [k0, k1, k2]
    x0 = jnp.full(x1.shape, k0, jnp.uint32)
    x1 = x1 + k1
    for i in range(5):
        for r in _ROT[i % 2]:
            x0 = x0 + x1
            x1 = (x1 << r) | (x1 >> (32 - r))
            x1 = x1 ^ x0
        x0 = x0 + ks[(i + 1) % 3]
        x1 = x1 + ks[(i + 2) % 3] + jnp.uint32(i + 1)
    return x0 ^ x1


def _bits_to_f01(bits):
    fb = (bits >> jnp.uint32(9)) | _ONEBITS
    return lax.bitcast_convert_type(fb, jnp.float32) - jnp.float32(1.0)


def _pack_body(x_ref, o_ref):
    x = x_ref[...]
    sh = lax.broadcasted_iota(jnp.int32, (x.shape[0], 32), 1)
    cols = []
    for j in range(8):
        w = x[:, 32 * j:32 * (j + 1)]
        cols.append(jnp.sum(w << sh, axis=1)[:, None])
    l = jnp.concatenate(cols, axis=1)
    o_ref[...] = jnp.concatenate([l, jnp.zeros_like(l)], axis=1)


def _pack(x):
    return pl.pallas_call(
        _pack_body,
        out_shape=jax.ShapeDtypeStruct((x.shape[0], _W16), jnp.int32),
    )(x)


def _make_sample(M, N, off_mult, k1, k2, rb, b0=0, nb=_B):
    grid_j = (2 * M) // rb

    def body(logits_ref, p_ref, samp_ref, inv_ref):
        j = pl.program_id(0)
        b = pl.program_id(1) + b0
        base = ((b * (2 * M) + j * rb) * N).astype(jnp.uint32)
        row = lax.broadcasted_iota(jnp.uint32, (rb, N), 0)
        col = lax.broadcasted_iota(jnp.uint32, (rb, N), 1)
        cnt = base + row * jnp.uint32(N) + col
        f = _bits_to_f01(_threefry_bits(k1, cnt))
        u = jnp.maximum(_TINY, f * (jnp.float32(1.0) - _TINY) + _TINY)
        g = -jnp.log(-jnp.log(u))
        v = g + logits_ref[...]
        rowmax = jnp.max(v, axis=1, keepdims=True)
        coli = lax.broadcasted_iota(jnp.int32, (rb, N), 1)
        cand = jnp.where(v == rowmax, coli, jnp.int32(N))
        samp = jnp.min(cand, axis=1) + b * jnp.int32(off_mult)
        samp_ref[...] = samp.reshape(1, 1, 1, rb)

        @pl.when(jnp.logical_and(j == 0, pl.program_id(1) == 0))
        def _():
            mi = lax.broadcasted_iota(jnp.uint32, (M, _W16), 0)
            f2 = _bits_to_f01(_threefry_bits(k2, mi))
            u2 = jnp.maximum(jnp.float32(0.0), f2)
            invw = jnp.where(u2 < p_ref[...], jnp.int32(-1), jnp.int32(0))
            inv_ref[...] = invw

    call = pl.pallas_call(
        body,
        grid=(grid_j, nb),
        in_specs=[
            pl.BlockSpec((rb, N), lambda j, b: (j, 0)),
            pl.BlockSpec((M, 1), lambda j, b: (0, 0)),
        ],
        out_specs=[
            pl.BlockSpec((1, 1, 1, rb), lambda j, b: (b, j, 0, 0)),
            pl.BlockSpec((M, _W16), lambda j, b: (0, 0)),
        ],
        out_shape=[
            jax.ShapeDtypeStruct((nb, grid_j, 1, rb), jnp.int32),
            jax.ShapeDtypeStruct((M, _W16), jnp.int32),
        ],
    )

    def run(logits2d, p_col):
        samp4, invw = call(logits2d, p_col)
        return samp4.reshape(nb, 2 * M), invw

    return run


_BSC = 10
_BCHUNK = 16384
_BUN = 8


def _make_sc_bits(rows, N, key):
    sz = rows * N
    szw = sz // 32
    n_outer = szw // _BCHUNK
    n_inner = _BCHUNK // (16 * _BUN)
    mesh = plsc.VectorSubcoreMesh(core_axis_name="c", subcore_axis_name="s")

    @functools.partial(
        pl.kernel,
        mesh=mesh,
        compiler_params=pltpu.CompilerParams(use_tc_tiling_on_sc=False),
        out_type=jax.ShapeDtypeStruct((sz,), jnp.uint32),
        scratch_types=[
            pltpu.VMEM((2, _BCHUNK), jnp.uint32),
            pltpu.SemaphoreType.DMA,
            pltpu.SemaphoreType.DMA,
        ],
    )
    def k(out_hbm, buf, sem0, sem1):
        w = lax.axis_index("s") * 2 + lax.axis_index("c")
        base = w * szw
        lane = lax.iota(jnp.uint32, 16)

        def outer(oi, carry):
            off = base + oi * _BCHUNK
            sel = lax.rem(oi, 2)

            def inner(ii, c2):
                o2 = ii * (16 * _BUN)
                c0 = jnp.uint32(off + o2) + lane
                for u in range(_BUN):
                    bits = _threefry_bits(key, c0 + jnp.uint32(16 * u))
                    buf[sel, pl.ds(o2 + 16 * u, 16)] = bits
                return c2

            @pl.when(oi >= 2)
            def _():
                @pl.when(sel == 0)
                def _():
                    pltpu.make_async_copy(
                        out_hbm.at[pl.ds(0, _BCHUNK)], buf.at[0], sem0).wait()

                @pl.when(sel == 1)
                def _():
                    pltpu.make_async_copy(
                        out_hbm.at[pl.ds(0, _BCHUNK)], buf.at[1], sem1).wait()

            lax.fori_loop(0, n_inner, inner, 0)

            @pl.when(sel == 0)
            def _():
                pltpu.async_copy(buf.at[0], out_hbm.at[pl.ds(off, _BCHUNK)], sem0)

            @pl.when(sel == 1)
            def _():
                pltpu.async_copy(buf.at[1], out_hbm.at[pl.ds(off, _BCHUNK)], sem1)

            return carry

        lax.fori_loop(0, n_outer, outer, 0)
        pltpu.make_async_copy(out_hbm.at[pl.ds(0, _BCHUNK)], buf.at[0], sem0).wait()
        pltpu.make_async_copy(out_hbm.at[pl.ds(0, _BCHUNK)], buf.at[1], sem1).wait()

    return k


def _make_sample_frombits(M, N, off_mult, rb):
    grid_j = (2 * M) // rb
    jpb = (2 * M) // rb

    def body(bits_ref, logits_ref, samp_ref):
        j = pl.program_id(0)
        b = pl.program_id(1)
        f = _bits_to_f01(bits_ref[...])
        u = jnp.maximum(_TINY, f * (jnp.float32(1.0) - _TINY) + _TINY)
        g = -jnp.log(-jnp.log(u))
        v = g + logits_ref[...]
        rowmax = jnp.max(v, axis=1, keepdims=True)
        coli = lax.broadcasted_iota(jnp.int32, (rb, N), 1)
        cand = jnp.where(v == rowmax, coli, jnp.int32(N))
        samp = jnp.min(cand, axis=1) + b * jnp.int32(off_mult)
        samp_ref[...] = samp.reshape(1, 1, 1, rb)

    call = pl.pallas_call(
        body,
        grid=(grid_j, _BSC),
        in_specs=[
            pl.BlockSpec((rb, N), lambda j, b: (b * jpb + j, 0)),
            pl.BlockSpec((rb, N), lambda j, b: (j, 0)),
        ],
        out_specs=pl.BlockSpec((1, 1, 1, rb), lambda j, b: (b, j, 0, 0)),
        out_shape=jax.ShapeDtypeStruct((_BSC, grid_j, 1, rb), jnp.int32),
    )

    def run(bits2d, logits2d):
        return call(bits2d, logits2d).reshape(_BSC, 2 * M)

    return run


_CH = 128


def _make_sc_layer(M):
    n_ch = M // _CH
    mesh = plsc.VectorSubcoreMesh(core_axis_name="c", subcore_axis_name="s")

    @functools.partial(
        pl.kernel,
        mesh=mesh,
        compiler_params=pltpu.CompilerParams(use_tc_tiling_on_sc=False),
        out_type=jax.ShapeDtypeStruct((_B, M, _W16), jnp.int32),
        scratch_types=[
            pltpu.VMEM((2, _CH), jnp.int32),
            pltpu.VMEM((_CH, _W16), jnp.int32),
            pltpu.VMEM((_CH, _W16), jnp.int32),
            pltpu.VMEM((_CH, _W16), jnp.int32),
            pltpu.VMEM((_CH, _W16), jnp.int32),
            pltpu.SemaphoreType.DMA,
            pltpu.SemaphoreType.DMA,
        ],
    )
    def k(src_hbm, samp_hbm, inv_hbm, out_hbm,
          idx_s, rows0, rows1, invv, outv, sem0, sem1):
        b = lax.axis_index("s") * 2 + lax.axis_index("c")

        def chunk(ci, carry):
            m0 = ci * _CH
            pltpu.sync_copy(samp_hbm.at[b, pl.ds(m0, _CH)], idx_s.at[0])
            pltpu.sync_copy(samp_hbm.at[b, pl.ds(M + m0, _CH)], idx_s.at[1])
            cp0 = pltpu.async_copy(src_hbm.at[idx_s.at[0]], rows0, sem0)
            cp1 = pltpu.async_copy(src_hbm.at[idx_s.at[1]], rows1, sem1)
            pltpu.sync_copy(inv_hbm.at[pl.ds(m0, _CH)], invv)
            cp0.wait()
            cp1.wait()

            def gate(i, c):
                a = rows0[i, :] & rows1[i, :]
                outv[i, :] = a ^ invv[i, :]
                return c

            lax.fori_loop(0, _CH, gate, 0)
            pltpu.sync_copy(outv, out_hbm.at[b, pl.ds(m0, _CH)])
            return carry

        lax.fori_loop(0, n_ch, chunk, 0)

    return k


def _unpack_body(y_ref, o_ref):
    y = y_ref[...]
    sh = lax.broadcasted_iota(jnp.int32, (y.shape[0], 32), 1)
    for j in range(8):
        lb = (y[:, j:j + 1] >> sh) & 1
        hb = (y[:, j + 8:j + 9] >> sh) & 1
        o_ref[:, 32 * j:32 * (j + 1)] = lb - 2 * hb


def _unpack(y, rb=2048):
    rows = y.shape[0]
    return pl.pallas_call(
        _unpack_body,
        grid=(rows // rb,),
        in_specs=[pl.BlockSpec((rb, _W16), lambda r: (r, 0))],
        out_specs=pl.BlockSpec((rb, 256), lambda r: (r, 0)),
        out_shape=jax.ShapeDtypeStruct((rows, 256), jnp.int32),
    )(y)


_SAMPLERS = [
    _make_sample(_MPL[l], _NPL[l], 0 if l == 0 else _NPL[l],
                 _KEYS[l][0], _KEYS[l][1], rb=512, b0=_BSC, nb=_B - _BSC)
    for l in range(4)
]
_SC_BITS = [
    _make_sc_bits(_BSC * 2 * _MPL[l], _NPL[l], _KEYS[l][0]) for l in range(4)
]
_SAMPLERS_FB = [
    _make_sample_frombits(_MPL[l], _NPL[l], 0 if l == 0 else _NPL[l], rb=512)
    for l in range(4)
]
_SC_LAYERS = [_make_sc_layer(_MPL[l]) for l in range(4)]


def kernel(input_bitarrays, batch_size,
           adj_logits_0, invert_logits_0,
           adj_logits_1, invert_logits_1,
           adj_logits_2, invert_logits_2,
           adj_logits_3, invert_logits_3):
    adjs = [adj_logits_0, adj_logits_1, adj_logits_2, adj_logits_3]
    invs = [invert_logits_0, invert_logits_1, invert_logits_2, invert_logits_3]

    bits = [_SC_BITS[l]().reshape(_BSC * 2 * _MPL[l], _NPL[l]) for l in range(4)]

    samps, invws = [], []
    for l in range(4):
        M, N = _MPL[l], _NPL[l]
        logits2d = adjs[l].reshape(2 * M, N)
        p_col = jax.nn.sigmoid(invs[l]).reshape(M, 1)
        s_hi, iw = _SAMPLERS[l](logits2d, p_col)
        s_lo = _SAMPLERS_FB[l](bits[l], logits2d)
        samps.append(jnp.concatenate([s_lo, s_hi], axis=0))
        invws.append(iw)

    x = _pack(input_bitarrays)
    for l in range(4):
        y = _SC_LAYERS[l](x, samps[l], invws[l])
        x = y.reshape(_B * _MPL[l], _W16)

    out = _unpack(x)
    return out.reshape(_B, _MPL[3], 256) + (batch_size - batch_size)
